# baseline (speedup 1.0000x reference)
import functools

import jax
import jax.numpy as jnp
from jax import lax
from jax.experimental import pallas as pl
from jax.experimental.pallas import tpu as pltpu
from jax.experimental.pallas import tpu_sc as plsc


def _make_sc_broadcast(batch, row_elems):
    info = plsc.get_sparse_core_info()
    num_workers = info.num_cores * info.num_subcores
    b_per_w = batch // num_workers
    rep = 8
    assert batch % num_workers == 0 and b_per_w % rep == 0
    n_stores = b_per_w // rep

    mesh = plsc.VectorSubcoreMesh(core_axis_name="c", subcore_axis_name="s")

    @functools.partial(
        pl.kernel,
        mesh=mesh,
        out_type=jax.ShapeDtypeStruct((batch, row_elems), jnp.float32),
        scratch_types=[
            pltpu.VMEM((rep, row_elems), jnp.float32),
            pltpu.SemaphoreType.DMA,
            pltpu.SemaphoreType.DMA,
        ],
    )
    def sc_broadcast(tbl_hbm, out_hbm, buf_v, in_sem, out_sem):
        wid = lax.axis_index("s") * info.num_cores + lax.axis_index("c")
        base = wid * b_per_w
        loads = [
            pltpu.async_copy(tbl_hbm, buf_v.at[i], in_sem) for i in range(rep)
        ]
        for cp in loads:
            cp.wait()
        stores = [
            pltpu.async_copy(
                buf_v, out_hbm.at[pl.ds(base + j * rep, rep)], out_sem
            )
            for j in range(n_stores)
        ]
        for cp in stores:
            cp.wait()

    return sc_broadcast


def kernel(sequence, pos_table):
    batch, seq_len = sequence.shape
    hidden = pos_table.shape[1]
    row_elems = seq_len * hidden
    flat = pos_table[:seq_len].reshape(row_elems)
    out = _make_sc_broadcast(batch, row_elems)(flat)
    return out.reshape(batch, seq_len, hidden)


# speedup vs baseline: 1.1791x; 1.1791x over previous
import jax
import jax.numpy as jnp
from jax.experimental import pallas as pl


def _row_broadcast_body(tbl_ref, out_ref):
    out_ref[...] = jnp.broadcast_to(tbl_ref[...], out_ref.shape)


def kernel(sequence, pos_table):
    batch, seq_len = sequence.shape
    hidden = pos_table.shape[1]
    row_elems = seq_len * hidden
    flat = pos_table[:seq_len].reshape(1, row_elems)
    bb, wb = 512, 3200
    out = pl.pallas_call(
        _row_broadcast_body,
        grid=(batch // bb, row_elems // wb),
        in_specs=[pl.BlockSpec((1, wb), lambda i, j: (0, j))],
        out_specs=pl.BlockSpec((bb, wb), lambda i, j: (i, j)),
        out_shape=jax.ShapeDtypeStruct((batch, row_elems), jnp.float32),
    )(flat)
    return out.reshape(batch, seq_len, hidden)
